# baseline (speedup 1.0000x reference)
import functools

import jax
import jax.numpy as jnp
from jax import lax
from jax.experimental import pallas as pl
from jax.experimental.pallas import tpu as pltpu
from jax.experimental.pallas import tpu_sc as plsc

N = 50000
E = 800000
ND = 16
ED = 16
MID = 20
NUM_LAYER = 12

N_PAD = 51200
E_PAD = 819200
NW = 32
EW = E_PAD // NW

C_EDGE = 1600
N_VALID_STEPS = E // C_EDGE
C_NODE = 1600

NODE_SLICE = N_PAD // 16

_mesh = plsc.VectorSubcoreMesh(core_axis_name="c", subcore_axis_name="s")
_f32 = jnp.float32
_SC_PARAMS = pltpu.CompilerParams(use_tc_tiling_on_sc=False)


def _mm(a, b):
  return jax.lax.dot(a, b, precision=jax.lax.Precision.HIGHEST)


def _make_gather2(sub, nsub):
  k128 = sub // 128

  @functools.partial(
      pl.kernel, mesh=_mesh,
      out_type=(jax.ShapeDtypeStruct((E_PAD, 16), _f32),
                jax.ShapeDtypeStruct((E_PAD, 16), _f32)),
      scratch_types=[pltpu.VMEM((128,), jnp.int32),
                     pltpu.VMEM((sub, 16), _f32),
                     pltpu.SemaphoreType.DMA],
      compiler_params=_SC_PARAMS,
  )
  def gk(tab_a, tab_b, idx_a, idx_b, out_a, out_b, idx_v, rows_v, sem):
    wid = lax.axis_index("s") * 2 + lax.axis_index("c")
    base = wid * EW

    def do_side(tab, idx1, out):
      def body(s, carry):
        b0 = pl.multiple_of(base + s * sub, 1024)
        for j in range(k128):
          pltpu.sync_copy(idx1.at[pl.ds(b0 + j * 128, 128)], idx_v)
          pltpu.async_copy(tab.at[idx_v],
                           rows_v.at[pl.ds(j * 128, 128)], sem).wait()
        pltpu.sync_copy(rows_v, out.at[pl.ds(b0, sub)])
        return carry
      lax.fori_loop(0, nsub, body, 0)

    do_side(tab_a, idx_a, out_a)
    do_side(tab_b, idx_b, out_b)

  return gk


_gather16 = _make_gather2(5120, 5)


def _make_scatter2pass(sub):
  k128 = sub // 128
  nsub = (E_PAD // 16) // sub

  @functools.partial(
      pl.kernel, mesh=_mesh,
      out_type=(jax.ShapeDtypeStruct((N_PAD, 16), _f32),) * 4,
      scratch_types=[pltpu.VMEM((128,), jnp.int32),
                     pltpu.VMEM((sub, 16), _f32),
                     pltpu.VMEM_SHARED((N_PAD, 16), _f32),
                     pltpu.SemaphoreType.DMA],
      compiler_params=_SC_PARAMS,
  )
  def sk(pa, pb, fa, fb, dsti, srci, zeros_hbm,
         spa, spb, sfa, sfb, idx_v, rows_v, accum, sem):
    cid = lax.axis_index("c")
    sid = lax.axis_index("s")
    r0 = pl.multiple_of(sid * NODE_SLICE, 1024)

    def zero_my_slice():
      pltpu.sync_copy(zeros_hbm, accum.at[pl.ds(r0, NODE_SLICE)])

    def side(data_hbm, idx1_hbm):
      def body(s, carry):
        b0 = pl.multiple_of(sid * (E_PAD // 16) + s * sub, 1024)
        pltpu.sync_copy(data_hbm.at[pl.ds(b0, sub)], rows_v)
        for j in range(k128):
          pltpu.sync_copy(idx1_hbm.at[pl.ds(b0 + j * 128, 128)], idx_v)
          pltpu.sync_copy(rows_v.at[pl.ds(j * 128, 128)],
                          accum.at[idx_v], add=True)
        return carry
      lax.fori_loop(0, nsub, body, 0)

    def one_pass(data_hbm, idx1_hbm, out):
      zero_my_slice()
      plsc.subcore_barrier()
      side(data_hbm, idx1_hbm)
      plsc.subcore_barrier()
      pltpu.sync_copy(accum.at[pl.ds(r0, NODE_SLICE)],
                      out.at[pl.ds(r0, NODE_SLICE)])
      plsc.subcore_barrier()

    @pl.when(cid == 0)
    def _():
      one_pass(pa, dsti, spa)
      one_pass(pb, dsti, spb)

    @pl.when(cid == 1)
    def _():
      one_pass(fa, srci, sfa)
      one_pass(fb, srci, sfb)

  return sk


_scatter2 = _make_scatter2pass(2048)


def _make_scatter_cnt(sub):
  k128 = sub // 128
  nsub = (E_PAD // 16) // sub

  @functools.partial(
      pl.kernel, mesh=_mesh,
      out_type=(jax.ShapeDtypeStruct((N_PAD, 16), _f32),) * 2,
      scratch_types=[pltpu.VMEM((128,), jnp.int32),
                     pltpu.VMEM((sub, 16), _f32),
                     pltpu.VMEM_SHARED((N_PAD, 16), _f32),
                     pltpu.SemaphoreType.DMA],
      compiler_params=_SC_PARAMS,
  )
  def ck(ones_hbm, dsti, srci, zeros_hbm, cd_out, cs_out,
         idx_v, rows_v, accum, sem):
    cid = lax.axis_index("c")
    sid = lax.axis_index("s")
    r0 = pl.multiple_of(sid * NODE_SLICE, 1024)
    pltpu.sync_copy(zeros_hbm, accum.at[pl.ds(r0, NODE_SLICE)])
    plsc.subcore_barrier()

    def side(idx1_hbm):
      def body(s, carry):
        b0 = pl.multiple_of(sid * (E_PAD // 16) + s * sub, 1024)
        pltpu.sync_copy(ones_hbm.at[pl.ds(b0, sub)], rows_v)
        for j in range(k128):
          pltpu.sync_copy(idx1_hbm.at[pl.ds(b0 + j * 128, 128)], idx_v)
          pltpu.sync_copy(rows_v.at[pl.ds(j * 128, 128)],
                          accum.at[idx_v], add=True)
        return carry
      lax.fori_loop(0, nsub, body, 0)

    @pl.when(cid == 0)
    def _():
      side(dsti)

    @pl.when(cid == 1)
    def _():
      side(srci)

    plsc.subcore_barrier()

    @pl.when(cid == 0)
    def _():
      pltpu.sync_copy(accum.at[pl.ds(r0, NODE_SLICE)],
                      cd_out.at[pl.ds(r0, NODE_SLICE)])

    @pl.when(cid == 1)
    def _():
      pltpu.sync_copy(accum.at[pl.ds(r0, NODE_SLICE)],
                      cs_out.at[pl.ds(r0, NODE_SLICE)])

  return ck


_scatter_cnt = _make_scatter_cnt(2048)


def _edge_body(mi_ref, mj_ref, h_ref, eiba_ref, eibb_ref, eica_ref, eicb_ref,
               we1_ref, be1_ref, we2_ref, be2_ref, wp1_ref, wf1_ref,
               hout_ref, pa_ref, pb_ref, fa_ref, fb_ref):
  step = pl.program_id(0)
  mi = mi_ref[...]
  mj = mj_ref[...]
  h = h_ref[...]
  t = jnp.maximum(_mm(mi, we1_ref[0:16]) + _mm(mj, we1_ref[16:32])
                  + _mm(h, we1_ref[32:48]) + be1_ref[...], 0.0)
  hn = _mm(t, we2_ref[...]) + be2_ref[...]
  hout_ref[...] = hn
  pre_pa = _mm(mi, wp1_ref[0:16, 0:16]) + _mm(hn, wp1_ref[16:32, 0:16]) \
      + eiba_ref[...]
  pre_pb = _mm(mi, wp1_ref[0:16, 16:32]) + _mm(hn, wp1_ref[16:32, 16:32]) \
      + eibb_ref[...]
  pre_fa = _mm(mj, wf1_ref[0:16, 0:16]) + _mm(hn, wf1_ref[16:32, 0:16]) \
      + eica_ref[...]
  pre_fb = _mm(mj, wf1_ref[0:16, 16:32]) + _mm(hn, wf1_ref[16:32, 16:32]) \
      + eicb_ref[...]
  valid = (step < N_VALID_STEPS).astype(_f32)
  pa_ref[...] = jnp.maximum(pre_pa, 0.0) * valid
  pb_ref[...] = jnp.maximum(pre_pb, 0.0) * valid
  fa_ref[...] = jnp.maximum(pre_fa, 0.0) * valid
  fb_ref[...] = jnp.maximum(pre_fb, 0.0) * valid


def _node_body(spa_ref, spb_ref, sfa_ref, sfb_ref, cd_ref, cs_ref,
               wp2_ref, bp2_ref, wf2_ref, bf2_ref,
               wv1_ref, bv1_ref, wv2_ref, bv2_ref, m_ref):
  cd = cd_ref[:, 0:1]
  cs = cs_ref[:, 0:1]
  rcd = 1.0 / jnp.maximum(cd, 1.0)
  rcs = 1.0 / jnp.maximum(cs, 1.0)
  af = (_mm(spa_ref[...] * rcd, wp2_ref[0:16])
        + _mm(spb_ref[...] * rcd, wp2_ref[16:32])
        + jnp.minimum(cd, 1.0) * bp2_ref[...])
  ap = (_mm(sfa_ref[...] * rcs, wf2_ref[0:16])
        + _mm(sfb_ref[...] * rcs, wf2_ref[16:32])
        + jnp.minimum(cs, 1.0) * bf2_ref[...])
  x = jnp.maximum(_mm(af, wv1_ref[0:16]) + _mm(ap, wv1_ref[16:32])
                  + bv1_ref[...], 0.0)
  m_ref[...] = _mm(x, wv2_ref[...]) + bv2_ref[...]


def _prep_body(m_ref, b3a_ref, bba_ref, b3b_ref, bbb_ref,
               c3a_ref, bca_ref, c3b_ref, bcb_ref,
               iba_ref, ibb_ref, ica_ref, icb_ref):
  m = m_ref[...]
  iba_ref[...] = _mm(m, b3a_ref[...]) + bba_ref[...]
  ibb_ref[...] = _mm(m, b3b_ref[...]) + bbb_ref[...]
  ica_ref[...] = _mm(m, c3a_ref[...]) + bca_ref[...]
  icb_ref[...] = _mm(m, c3b_ref[...]) + bcb_ref[...]


def _final_body(h_ref, wo1_ref, bo1_ref, wo2_ref, bo2_ref, o_ref):
  t = jnp.maximum(_mm(h_ref[...], wo1_ref[...]) + bo1_ref[...], 0.0)
  o_ref[...] = _mm(t, wo2_ref[...]) + bo2_ref[...]


def _full(shape):
  nd = len(shape)
  return pl.BlockSpec(shape, lambda i: (0,) * nd)


def _rows(c, w):
  return pl.BlockSpec((c, w), lambda i: (i, 0))


def kernel(M, H, edge_index, We1, be1, We2, be2, Wp1, bp1, Wp2, bp2,
           Wf1, bf1, Wf2, bf2, Wv1, bv1, Wv2, bv2, Wo1, bo1, Wo2, bo2):
  f32 = _f32
  M_pad = jnp.pad(M, ((0, N_PAD - N), (0, 0)))
  H_pad = jnp.pad(H, ((0, E_PAD - E), (0, 0)))
  src = jnp.pad(edge_index[0], (0, E_PAD - E))
  dst = jnp.pad(edge_index[1], (0, E_PAD - E))

  def padw(w):
    return jnp.pad(w, ((0, 0), (0, 16 - w.shape[1])))

  def padb(b):
    return jnp.pad(b, (0, 16 - b.shape[0])).reshape(1, 16)

  wp1q = jnp.concatenate(
      [jnp.concatenate([Wp1[0:16, 0:16], padw(Wp1[0:16, 16:20])], 1),
       jnp.concatenate([Wp1[16:32, 0:16], padw(Wp1[16:32, 16:20])], 1)], 0)
  wf1q = jnp.concatenate(
      [jnp.concatenate([Wf1[0:16, 0:16], padw(Wf1[0:16, 16:20])], 1),
       jnp.concatenate([Wf1[16:32, 0:16], padw(Wf1[16:32, 16:20])], 1)], 0)
  b3a, b3b = Wp1[32:48, 0:16], padw(Wp1[32:48, 16:20])
  c3a, c3b = Wf1[32:48, 0:16], padw(Wf1[32:48, 16:20])
  bba, bbb = bp1[0:16].reshape(1, 16), padb(bp1[16:20])
  bca, bcb = bf1[0:16].reshape(1, 16), padb(bf1[16:20])
  wp2q = jnp.concatenate([Wp2[0:16], jnp.pad(Wp2[16:20], ((0, 12), (0, 0)))],
                         0)
  wf2q = jnp.concatenate([Wf2[0:16], jnp.pad(Wf2[16:20], ((0, 12), (0, 0)))],
                         0)
  be1r = be1.reshape(1, MID)
  be2r = be2.reshape(1, ED)
  bp2r = bp2.reshape(1, ND)
  bf2r = bf2.reshape(1, ND)
  bv1r = bv1.reshape(1, MID)
  bv2r = bv2.reshape(1, ND)
  bo1r = bo1.reshape(1, MID)
  bo2r = bo2.reshape(1, 2)
  zeros16 = jnp.zeros((NODE_SLICE, 16), f32)
  count_rows = jnp.zeros((E_PAD, 16), f32).at[:E, 0].set(1.0)

  cdt, cst = _scatter_cnt(count_rows, dst, src, zeros16)

  iba, ibb, ica, icb = pl.pallas_call(
      _prep_body,
      grid=(N_PAD // C_NODE,),
      in_specs=[_rows(C_NODE, 16)] + [_full((16, 16)), _full((1, 16))] * 4,
      out_specs=[_rows(C_NODE, 16)] * 4,
      out_shape=[jax.ShapeDtypeStruct((N_PAD, 16), f32)] * 4,
  )(M_pad, b3a, bba, b3b, bbb, c3a, bca, c3b, bcb)
  eiba, eibb = _gather16(iba, ibb, src, src)
  eica, eicb = _gather16(ica, icb, dst, dst)

  _tc_params = pltpu.CompilerParams(vmem_limit_bytes=110 * 1024 * 1024)
  edge_call = pl.pallas_call(
      _edge_body,
      compiler_params=_tc_params,
      grid=(E_PAD // C_EDGE,),
      in_specs=[_rows(C_EDGE, 16)] * 7
      + [_full((48, 20)), _full((1, 20)), _full((20, 16)), _full((1, 16)),
         _full((32, 32)), _full((32, 32))],
      out_specs=[_rows(C_EDGE, 16)] * 5,
      out_shape=[jax.ShapeDtypeStruct((E_PAD, 16), f32)] * 5,
  )

  node_call = pl.pallas_call(
      _node_body,
      compiler_params=_tc_params,
      grid=(N_PAD // C_NODE,),
      in_specs=[_rows(C_NODE, 16)] * 6
      + [_full((32, 16)), _full((1, 16)), _full((32, 16)), _full((1, 16)),
         _full((32, 20)), _full((1, 20)), _full((20, 16)), _full((1, 16))],
      out_specs=_rows(C_NODE, 16),
      out_shape=jax.ShapeDtypeStruct((N_PAD, 16), f32),
  )

  m_cur = M_pad
  h_cur = H_pad
  for _ in range(NUM_LAYER):
    mi, mj = _gather16(m_cur, m_cur, dst, src)
    h_cur, pa, pb, fa, fb = edge_call(mi, mj, h_cur, eiba, eibb, eica, eicb,
                                      We1, be1r, We2, be2r, wp1q, wf1q)
    spa, spb, sfa, sfb = _scatter2(pa, pb, fa, fb, dst, src, zeros16)
    m_cur = node_call(spa, spb, sfa, sfb, cdt, cst, wp2q, bp2r, wf2q, bf2r,
                      Wv1, bv1r, Wv2, bv2r)

  out = pl.pallas_call(
      _final_body,
      grid=(E // C_EDGE,),
      in_specs=[_rows(C_EDGE, 16), _full((16, 20)), _full((1, 20)),
                _full((20, 2)), _full((1, 2))],
      out_specs=_rows(C_EDGE, 2),
      out_shape=jax.ShapeDtypeStruct((E, 2), f32),
  )(h_cur, Wo1, bo1r, Wo2, bo2r)
  return out
